# Initial kernel scaffold; baseline (speedup 1.0000x reference)
import jax
import jax.numpy as jnp
from jax.experimental import pallas as pl


def kernel(x, adj_indices, adj_values, W1, b1, gamma1, beta1, W2, b2, gamma2, beta2, W3, b3):
    raise NotImplementedError("write your pallas kernel here")


# speedup vs baseline: 8.7999x; 8.7999x over previous
import functools

import jax
import jax.numpy as jnp
from jax import lax
from jax.experimental import pallas as pl
from jax.experimental.pallas import tpu as pltpu
from jax.experimental.pallas import tpu_sc as plsc

NC = 2
NS = 16
LANES = 16
DPAD = 128
EDGE_BATCH = 128


def _mm(x, w):
    n, kdim = x.shape
    dout = w.shape[0]
    br = 1000

    def body(x_ref, w_ref, o_ref):
        o_ref[...] = lax.dot_general(
            x_ref[...], w_ref[...], (((1,), (1,)), ((), ())),
            preferred_element_type=jnp.float32)

    return pl.pallas_call(
        body,
        grid=(n // br,),
        in_specs=[pl.BlockSpec((br, kdim), lambda i: (i, 0)),
                  pl.BlockSpec((dout, kdim), lambda i: (0, 0))],
        out_specs=pl.BlockSpec((br, dout), lambda i: (i, 0)),
        out_shape=jax.ShapeDtypeStruct((n, dout), jnp.float32),
    )(x, w)


def _fuse(part, w, gamma, beta, n, dk):
    dout = w.shape[0]

    def body(p_ref, w_ref, g_ref, bt_ref, o_ref):
        s = p_ref[0, :, :dk] + p_ref[1, :, :dk]
        m = jnp.mean(s, axis=0, keepdims=True)
        c = s - m
        v = jnp.mean(c * c, axis=0, keepdims=True)
        h = g_ref[...] * c * lax.rsqrt(v + 1e-5) + bt_ref[...]
        h = jnp.maximum(h, 0.0)
        o_ref[...] = lax.dot_general(
            h, w_ref[...], (((1,), (1,)), ((), ())),
            preferred_element_type=jnp.float32)

    return pl.pallas_call(
        body,
        grid=(1,),
        in_specs=[pl.BlockSpec((2, n, DPAD), lambda i: (0, 0, 0)),
                  pl.BlockSpec((dout, dk), lambda i: (0, 0)),
                  pl.BlockSpec((1, dk), lambda i: (0, 0)),
                  pl.BlockSpec((1, dk), lambda i: (0, 0))],
        out_specs=pl.BlockSpec((n, dout), lambda i: (0, 0)),
        out_shape=jax.ShapeDtypeStruct((n, dout), jnp.float32),
    )(part, w, gamma.reshape(1, dk), beta.reshape(1, dk))


def _final_add(part, b3p, n):
    def body(p_ref, b_ref, o_ref):
        o_ref[...] = p_ref[0] + p_ref[1] + b_ref[...]

    return pl.pallas_call(
        body,
        grid=(1,),
        in_specs=[pl.BlockSpec((2, n, DPAD), lambda i: (0, 0, 0)),
                  pl.BlockSpec((1, DPAD), lambda i: (0, 0))],
        out_specs=pl.BlockSpec((n, DPAD), lambda i: (0, 0)),
        out_shape=jax.ShapeDtypeStruct((n, DPAD), jnp.float32),
    )(part, b3p.reshape(1, DPAD))


@functools.lru_cache(maxsize=None)
def _make_spmm(n_pad, e_pad, nvec_scale):
    epw = e_pad // (NC * NS)
    nb = epw // EDGE_BATCH
    rpt = n_pad // NS
    mesh = plsc.VectorSubcoreMesh(core_axis_name="c", subcore_axis_name="s")

    @functools.partial(
        pl.kernel,
        out_type=jax.ShapeDtypeStruct((NC, n_pad, DPAD), jnp.float32),
        mesh=mesh,
        scratch_types=[
            pltpu.VMEM_SHARED((n_pad, DPAD), jnp.float32),
            pltpu.VMEM((EDGE_BATCH,), jnp.int32),
            pltpu.VMEM((EDGE_BATCH,), jnp.int32),
            pltpu.VMEM((EDGE_BATCH,), jnp.float32),
            pltpu.VMEM((EDGE_BATCH, DPAD), jnp.float32),
            pltpu.SemaphoreType.DMA,
        ],
    )
    def spmm(h_hbm, src_hbm, dst_hbm, val_hbm, zero_hbm, out_hbm,
             acc, src_v, dst_v, val_s, rows_v, sem):
        cid = lax.axis_index("c")
        sid = lax.axis_index("s")
        pltpu.sync_copy(zero_hbm.at[pl.ds(sid * rpt, rpt)],
                        acc.at[pl.ds(sid * rpt, rpt)])
        plsc.subcore_barrier()

        base = (cid * NS + sid) * epw

        @pl.loop(0, nb)
        def _batch(b):
            off = base + b * EDGE_BATCH
            pltpu.sync_copy(src_hbm.at[pl.ds(off, EDGE_BATCH)], src_v)
            pltpu.sync_copy(dst_hbm.at[pl.ds(off, EDGE_BATCH)], dst_v)
            pltpu.sync_copy(val_hbm.at[pl.ds(off, EDGE_BATCH)], val_s)
            pltpu.async_copy(h_hbm.at[src_v], rows_v, sem).wait()

            @pl.loop(0, EDGE_BATCH // LANES)
            def _grp(g):
                vvv = val_s[pl.ds(g * LANES, LANES)]
                for i in range(LANES):
                    j = g * LANES + i
                    vv = vvv[i]
                    for k in range(nvec_scale):
                        sl = pl.ds(k * LANES, LANES)
                        rows_v[j, sl] = rows_v[j, sl] * vv

            pltpu.sync_copy(rows_v, acc.at[dst_v], add=True)

        plsc.subcore_barrier()
        pltpu.sync_copy(acc.at[pl.ds(sid * rpt, rpt)],
                        out_hbm.at[cid, pl.ds(sid * rpt, rpt)])

    return spmm


def kernel(x, adj_indices, adj_values, W1, b1, gamma1, beta1,
           W2, b2, gamma2, beta2, W3, b3):
    n = x.shape[0]
    hid = W1.shape[0]
    e = adj_values.shape[0]
    group = NC * NS * EDGE_BATCH
    e_pad = ((e + group - 1) // group) * group
    pad = e_pad - e
    n_pad = ((n + NS * 8 - 1) // (NS * 8)) * (NS * 8)

    dst = jnp.concatenate([adj_indices[0], jnp.zeros((pad,), jnp.int32)])
    src = jnp.concatenate([adj_indices[1], jnp.zeros((pad,), jnp.int32)])
    val = jnp.concatenate([adj_values, jnp.zeros((pad,), jnp.float32)])
    zpad = jnp.zeros((n_pad, DPAD), jnp.float32)
    w1p = jnp.zeros((DPAD, x.shape[1]), jnp.float32).at[:hid, :].set(W1)
    w2p = jnp.zeros((DPAD, hid), jnp.float32).at[:hid, :].set(W2)
    w3p = jnp.zeros((DPAD, hid), jnp.float32).at[:W3.shape[0], :].set(W3)
    b3p = jnp.zeros((DPAD,), jnp.float32).at[:W3.shape[0]].set(b3)

    spmm64 = _make_spmm(n_pad, e_pad, hid // LANES)
    spmm16 = _make_spmm(n_pad, e_pad, 1)

    y1 = _mm(x, w1p)
    p1 = spmm64(y1, src, dst, val, zpad)
    y2 = _fuse(p1, w2p, gamma1, beta1, n, hid)
    p2 = spmm64(y2, src, dst, val, zpad)
    y3 = _fuse(p2, w3p, gamma2, beta2, n, hid)
    p3 = spmm16(y3, src, dst, val, zpad)
    out = _final_add(p3, b3p, n)
    return out[:, :W3.shape[0]]


# speedup vs baseline: 13.7266x; 1.5599x over previous
import functools

import jax
import jax.numpy as jnp
from jax import lax
from jax.experimental import pallas as pl
from jax.experimental.pallas import tpu as pltpu
from jax.experimental.pallas import tpu_sc as plsc

NC = 2
NS = 16
LANES = 16
EDGE_BATCH = 128


def _mm(x, w):
    n, kdim = x.shape
    dout = w.shape[0]
    br = 1000

    def body(x_ref, w_ref, o_ref):
        o_ref[...] = lax.dot_general(
            x_ref[...], w_ref[...], (((1,), (1,)), ((), ())),
            preferred_element_type=jnp.float32)

    return pl.pallas_call(
        body,
        grid=(n // br,),
        in_specs=[pl.BlockSpec((br, kdim), lambda i: (i, 0)),
                  pl.BlockSpec((dout, kdim), lambda i: (0, 0))],
        out_specs=pl.BlockSpec((br, dout), lambda i: (i, 0)),
        out_shape=jax.ShapeDtypeStruct((n, dout), jnp.float32),
    )(x, w)


def _fuse(part, w, gamma, beta, n):
    dk = part.shape[2]
    dout = w.shape[0]

    def body(p_ref, w_ref, g_ref, bt_ref, o_ref):
        s = p_ref[0] + p_ref[1]
        m = jnp.mean(s, axis=0, keepdims=True)
        c = s - m
        v = jnp.mean(c * c, axis=0, keepdims=True)
        h = g_ref[...] * c * lax.rsqrt(v + 1e-5) + bt_ref[...]
        h = jnp.maximum(h, 0.0)
        o_ref[...] = lax.dot_general(
            h, w_ref[...], (((1,), (1,)), ((), ())),
            preferred_element_type=jnp.float32)

    return pl.pallas_call(
        body,
        grid=(1,),
        in_specs=[pl.BlockSpec((2, n, dk), lambda i: (0, 0, 0)),
                  pl.BlockSpec((dout, dk), lambda i: (0, 0)),
                  pl.BlockSpec((1, dk), lambda i: (0, 0)),
                  pl.BlockSpec((1, dk), lambda i: (0, 0))],
        out_specs=pl.BlockSpec((n, dout), lambda i: (0, 0)),
        out_shape=jax.ShapeDtypeStruct((n, dout), jnp.float32),
    )(part, w, gamma.reshape(1, dk), beta.reshape(1, dk))


def _final_add(part, b3p, n):
    dk = part.shape[2]

    def body(p_ref, b_ref, o_ref):
        o_ref[...] = p_ref[0] + p_ref[1] + b_ref[...]

    return pl.pallas_call(
        body,
        grid=(1,),
        in_specs=[pl.BlockSpec((2, n, dk), lambda i: (0, 0, 0)),
                  pl.BlockSpec((1, dk), lambda i: (0, 0))],
        out_specs=pl.BlockSpec((n, dk), lambda i: (0, 0)),
        out_shape=jax.ShapeDtypeStruct((n, dk), jnp.float32),
    )(part, b3p.reshape(1, dk))


@functools.lru_cache(maxsize=None)
def _make_spmm(n_pad, dk, e_pad):
    epw = e_pad // (NC * NS)
    nb = epw // EDGE_BATCH
    rpt = n_pad // NS
    nvec = dk // LANES
    mesh = plsc.VectorSubcoreMesh(core_axis_name="c", subcore_axis_name="s")

    @functools.partial(
        pl.kernel,
        out_type=jax.ShapeDtypeStruct((NC, n_pad, dk), jnp.float32),
        mesh=mesh,
        compiler_params=pltpu.CompilerParams(use_tc_tiling_on_sc=False),
        scratch_types=[
            pltpu.VMEM_SHARED((n_pad, dk), jnp.float32),
            pltpu.VMEM((EDGE_BATCH,), jnp.int32),
            pltpu.VMEM((EDGE_BATCH,), jnp.int32),
            pltpu.VMEM((EDGE_BATCH,), jnp.float32),
            pltpu.VMEM((EDGE_BATCH, dk), jnp.float32),
            pltpu.SemaphoreType.DMA,
        ],
    )
    def spmm(h_hbm, src_hbm, dst_hbm, val_hbm, zero_hbm, out_hbm,
             acc, src_v, dst_v, val_v, rows_v, sem):
        cid = lax.axis_index("c")
        sid = lax.axis_index("s")
        pltpu.sync_copy(zero_hbm.at[pl.ds(sid * rpt, rpt)],
                        acc.at[pl.ds(sid * rpt, rpt)])
        plsc.subcore_barrier()

        base = (cid * NS + sid) * epw

        @pl.loop(0, nb)
        def _batch(b):
            off = base + b * EDGE_BATCH
            pltpu.sync_copy(src_hbm.at[pl.ds(off, EDGE_BATCH)], src_v)
            pltpu.sync_copy(dst_hbm.at[pl.ds(off, EDGE_BATCH)], dst_v)
            pltpu.sync_copy(val_hbm.at[pl.ds(off, EDGE_BATCH)], val_v)
            pltpu.async_copy(h_hbm.at[src_v], rows_v, sem).wait()

            @pl.loop(0, EDGE_BATCH // LANES)
            def _grp(g):
                vvv = val_v[pl.ds(g * LANES, LANES)]
                for i in range(LANES):
                    j = g * LANES + i
                    vv = vvv[i]
                    for k in range(nvec):
                        sl = pl.ds(k * LANES, LANES)
                        rows_v[j, sl] = rows_v[j, sl] * vv

            pltpu.sync_copy(rows_v, acc.at[dst_v], add=True)

        plsc.subcore_barrier()
        pltpu.sync_copy(acc.at[pl.ds(sid * rpt, rpt)],
                        out_hbm.at[cid, pl.ds(sid * rpt, rpt)])

    return spmm


def kernel(x, adj_indices, adj_values, W1, b1, gamma1, beta1,
           W2, b2, gamma2, beta2, W3, b3):
    n = x.shape[0]
    hid = W1.shape[0]
    dlast = 16
    e = adj_values.shape[0]
    group = NC * NS * EDGE_BATCH
    e_pad = ((e + group - 1) // group) * group
    pad = e_pad - e
    n_pad = ((n + NS * 8 - 1) // (NS * 8)) * (NS * 8)

    dst = jnp.concatenate([adj_indices[0], jnp.zeros((pad,), jnp.int32)])
    src = jnp.concatenate([adj_indices[1], jnp.zeros((pad,), jnp.int32)])
    val = jnp.concatenate([adj_values, jnp.zeros((pad,), jnp.float32)])
    zhid = jnp.zeros((n_pad, hid), jnp.float32)
    zlast = jnp.zeros((n_pad, dlast), jnp.float32)
    w3p = jnp.zeros((dlast, hid), jnp.float32).at[:W3.shape[0], :].set(W3)
    b3p = jnp.zeros((dlast,), jnp.float32).at[:W3.shape[0]].set(b3)

    spmm_h = _make_spmm(n_pad, hid, e_pad)
    spmm_l = _make_spmm(n_pad, dlast, e_pad)

    y1 = _mm(x, W1)
    p1 = spmm_h(y1, src, dst, val, zhid)
    y2 = _fuse(p1, W2, gamma1, beta1, n)
    p2 = spmm_h(y2, src, dst, val, zhid)
    y3 = _fuse(p2, w3p, gamma2, beta2, n)
    p3 = spmm_l(y3, src, dst, val, zlast)
    out = _final_add(p3, b3p, n)
    return out[:, :W3.shape[0]]


# speedup vs baseline: 16.3346x; 1.1900x over previous
import functools

import jax
import jax.numpy as jnp
from jax import lax
from jax.experimental import pallas as pl
from jax.experimental.pallas import tpu as pltpu
from jax.experimental.pallas import tpu_sc as plsc

NC = 2
NS = 16
LANES = 16
EDGE_BATCH = 128


def _mm(x, w):
    n, kdim = x.shape
    dout = w.shape[0]
    br = 1000

    def body(x_ref, w_ref, o_ref):
        o_ref[...] = lax.dot_general(
            x_ref[...], w_ref[...], (((1,), (1,)), ((), ())),
            preferred_element_type=jnp.float32)

    return pl.pallas_call(
        body,
        grid=(n // br,),
        in_specs=[pl.BlockSpec((br, kdim), lambda i: (i, 0)),
                  pl.BlockSpec((dout, kdim), lambda i: (0, 0))],
        out_specs=pl.BlockSpec((br, dout), lambda i: (i, 0)),
        out_shape=jax.ShapeDtypeStruct((n, dout), jnp.float32),
    )(x, w)


def _fuse(part, w, gamma, beta, n):
    dk = part.shape[2]
    dout = w.shape[0]

    def body(p_ref, w_ref, g_ref, bt_ref, o_ref):
        s = p_ref[0] + p_ref[1]
        m = jnp.mean(s, axis=0, keepdims=True)
        c = s - m
        v = jnp.mean(c * c, axis=0, keepdims=True)
        h = g_ref[...] * c * lax.rsqrt(v + 1e-5) + bt_ref[...]
        h = jnp.maximum(h, 0.0)
        o_ref[...] = lax.dot_general(
            h, w_ref[...], (((1,), (1,)), ((), ())),
            preferred_element_type=jnp.float32)

    return pl.pallas_call(
        body,
        grid=(1,),
        in_specs=[pl.BlockSpec((2, n, dk), lambda i: (0, 0, 0)),
                  pl.BlockSpec((dout, dk), lambda i: (0, 0)),
                  pl.BlockSpec((1, dk), lambda i: (0, 0)),
                  pl.BlockSpec((1, dk), lambda i: (0, 0))],
        out_specs=pl.BlockSpec((n, dout), lambda i: (0, 0)),
        out_shape=jax.ShapeDtypeStruct((n, dout), jnp.float32),
    )(part, w, gamma.reshape(1, dk), beta.reshape(1, dk))


def _final_add(part, b3p, n):
    dk = part.shape[2]

    def body(p_ref, b_ref, o_ref):
        o_ref[...] = p_ref[0] + p_ref[1] + b_ref[...]

    return pl.pallas_call(
        body,
        grid=(1,),
        in_specs=[pl.BlockSpec((2, n, dk), lambda i: (0, 0, 0)),
                  pl.BlockSpec((1, dk), lambda i: (0, 0))],
        out_specs=pl.BlockSpec((n, dk), lambda i: (0, 0)),
        out_shape=jax.ShapeDtypeStruct((n, dk), jnp.float32),
    )(part, b3p.reshape(1, dk))


@functools.lru_cache(maxsize=None)
def _make_spmm(n_pad, dk, nb):
    rpt = n_pad // NS
    nvec = dk // LANES
    ngrp = EDGE_BATCH // LANES
    assert nb >= 2 and nb % 2 == 0
    mesh = plsc.VectorSubcoreMesh(core_axis_name="c", subcore_axis_name="s")

    @functools.partial(
        pl.kernel,
        out_type=jax.ShapeDtypeStruct((NC, n_pad, dk), jnp.float32),
        mesh=mesh,
        compiler_params=pltpu.CompilerParams(
            use_tc_tiling_on_sc=False, needs_layout_passes=False),
        scratch_types=[
            pltpu.VMEM_SHARED((n_pad, dk), jnp.float32),
            pltpu.VMEM((2, 4, EDGE_BATCH), jnp.int32),
            pltpu.VMEM((2, EDGE_BATCH, dk), jnp.float32),
            pltpu.SemaphoreType.DMA,
            pltpu.SemaphoreType.DMA,
            pltpu.SemaphoreType.DMA,
            pltpu.SemaphoreType.DMA,
        ],
    )
    def spmm(h_hbm, edata_hbm, zero_hbm, out_hbm,
             acc, e_v, rows, se0, se1, sg0, sg1):
        cid = lax.axis_index("c")
        sid = lax.axis_index("s")
        wid = cid * NS + sid
        se = (se0, se1)
        sg = (sg0, sg1)

        pltpu.sync_copy(zero_hbm.at[pl.ds(sid * rpt, rpt)],
                        acc.at[pl.ds(sid * rpt, rpt)])
        plsc.subcore_barrier()

        def gather_start(p):
            pltpu.async_copy(h_hbm.at[e_v.at[p].at[0]], rows.at[p], sg[p])

        def gather_wait(p):
            pltpu.make_async_copy(
                h_hbm.at[e_v.at[p].at[0]], rows.at[p], sg[p]).wait()

        def edata_start(p, bb):
            pltpu.async_copy(edata_hbm.at[wid, bb], e_v.at[p], se[p])

        def edata_wait(p):
            pltpu.make_async_copy(
                edata_hbm.at[wid, 0], e_v.at[p], se[p]).wait()

        def step(bb, p):
            q = 1 - p
            edata_wait(q)
            gather_start(q)
            gather_wait(p)
            ep = e_v.at[p]
            rp = rows.at[p]

            @pl.loop(0, ngrp)
            def _grp(g):
                vvv = plsc.bitcast(ep[2, pl.ds(g * LANES, LANES)],
                                   jnp.float32)
                for i in range(LANES):
                    j = g * LANES + i
                    vv = vvv[i]
                    for k in range(nvec):
                        sl = pl.ds(k * LANES, LANES)
                        rp[j, sl] = rp[j, sl] * vv

            pltpu.sync_copy(rp, acc.at[ep.at[1]], add=True)
            edata_start(p, bb + 2)

        pltpu.async_copy(edata_hbm.at[wid, 0], e_v.at[0], se[0]).wait()
        gather_start(0)
        edata_start(1, 1)

        @pl.loop(0, nb, step=2)
        def _pair(b):
            step(b, 0)
            step(b + 1, 1)

        edata_wait(1)
        gather_wait(0)

        plsc.subcore_barrier()
        pltpu.sync_copy(acc.at[pl.ds(sid * rpt, rpt)],
                        out_hbm.at[cid, pl.ds(sid * rpt, rpt)])

    return spmm


def kernel(x, adj_indices, adj_values, W1, b1, gamma1, beta1,
           W2, b2, gamma2, beta2, W3, b3):
    n = x.shape[0]
    hid = W1.shape[0]
    dlast = 16
    e = adj_values.shape[0]
    group = NC * NS * EDGE_BATCH
    e_pad = ((e + group - 1) // group) * group
    pad = e_pad - e
    n_pad = ((n + NS * 8 - 1) // (NS * 8)) * (NS * 8)

    dst = jnp.concatenate([adj_indices[0], jnp.zeros((pad,), jnp.int32)])
    src = jnp.concatenate([adj_indices[1], jnp.zeros((pad,), jnp.int32)])
    val = jnp.concatenate([adj_values, jnp.zeros((pad,), jnp.float32)])
    nw = NC * NS
    epw = e_pad // nw
    nb = epw // EDGE_BATCH

    def _tile(a):
        a = a.reshape(nw, epw)
        a = jnp.concatenate(
            [a, jnp.zeros((nw, 2 * EDGE_BATCH), jnp.int32)], axis=1)
        return a.reshape(nw, nb + 2, EDGE_BATCH)

    edata = jnp.stack(
        [_tile(src), _tile(dst), _tile(lax.bitcast_convert_type(val, jnp.int32)),
         _tile(jnp.zeros((e_pad,), jnp.int32))], axis=2)

    zhid = jnp.zeros((n_pad, hid), jnp.float32)
    zlast = jnp.zeros((n_pad, dlast), jnp.float32)
    w3p = jnp.zeros((dlast, hid), jnp.float32).at[:W3.shape[0], :].set(W3)
    b3p = jnp.zeros((dlast,), jnp.float32).at[:W3.shape[0]].set(b3)

    spmm_h = _make_spmm(n_pad, hid, nb)
    spmm_l = _make_spmm(n_pad, dlast, nb)

    y1 = _mm(x, W1)
    p1 = spmm_h(y1, edata, zhid)
    y2 = _fuse(p1, W2, gamma1, beta1, n)
    p2 = spmm_h(y2, edata, zhid)
    y3 = _fuse(p2, w3p, gamma2, beta2, n)
    p3 = spmm_l(y3, edata, zlast)
    out = _final_add(p3, b3p, n)
    return out[:, :W3.shape[0]]


# speedup vs baseline: 29.6378x; 1.8144x over previous
import functools

import jax
import jax.numpy as jnp
from jax import lax
from jax.experimental import pallas as pl
from jax.experimental.pallas import tpu as pltpu
from jax.experimental.pallas import tpu_sc as plsc

NC = 2
NS = 16
LANES = 16
EDGE_BATCH = 128


def _mm(x, w):
    n, kdim = x.shape
    dout = w.shape[0]
    br = 1000

    def body(x_ref, w_ref, o_ref):
        o_ref[...] = lax.dot_general(
            x_ref[...], w_ref[...], (((1,), (1,)), ((), ())),
            preferred_element_type=jnp.float32)

    return pl.pallas_call(
        body,
        grid=(n // br,),
        in_specs=[pl.BlockSpec((br, kdim), lambda i: (i, 0)),
                  pl.BlockSpec((dout, kdim), lambda i: (0, 0))],
        out_specs=pl.BlockSpec((br, dout), lambda i: (i, 0)),
        out_shape=jax.ShapeDtypeStruct((n, dout), jnp.float32),
    )(x, w)


def _fuse(part, w, gamma, beta, n):
    dk = part.shape[2]
    dout = w.shape[0]

    def body(p_ref, w_ref, g_ref, bt_ref, o_ref):
        s = p_ref[0] + p_ref[1]
        m = jnp.mean(s, axis=0, keepdims=True)
        c = s - m
        v = jnp.mean(c * c, axis=0, keepdims=True)
        h = g_ref[...] * c * lax.rsqrt(v + 1e-5) + bt_ref[...]
        h = jnp.maximum(h, 0.0)
        o_ref[...] = lax.dot_general(
            h, w_ref[...], (((1,), (1,)), ((), ())),
            preferred_element_type=jnp.float32)

    return pl.pallas_call(
        body,
        grid=(1,),
        in_specs=[pl.BlockSpec((2, n, dk), lambda i: (0, 0, 0)),
                  pl.BlockSpec((dout, dk), lambda i: (0, 0)),
                  pl.BlockSpec((1, dk), lambda i: (0, 0)),
                  pl.BlockSpec((1, dk), lambda i: (0, 0))],
        out_specs=pl.BlockSpec((n, dout), lambda i: (0, 0)),
        out_shape=jax.ShapeDtypeStruct((n, dout), jnp.float32),
    )(part, w, gamma.reshape(1, dk), beta.reshape(1, dk))


def _final_add(part, b3p, n):
    dk = part.shape[2]

    def body(p_ref, b_ref, o_ref):
        o_ref[...] = p_ref[0] + p_ref[1] + b_ref[...]

    return pl.pallas_call(
        body,
        grid=(1,),
        in_specs=[pl.BlockSpec((2, n, dk), lambda i: (0, 0, 0)),
                  pl.BlockSpec((1, dk), lambda i: (0, 0))],
        out_specs=pl.BlockSpec((n, dk), lambda i: (0, 0)),
        out_shape=jax.ShapeDtypeStruct((n, dk), jnp.float32),
    )(part, b3p.reshape(1, dk))


@functools.lru_cache(maxsize=None)
def _make_spmm(n_pad, dk, nb):
    rpt = n_pad // NS
    nvec = dk // LANES
    ngrp = EDGE_BATCH // LANES
    assert nb >= 2 and nb % 2 == 0
    mesh = plsc.VectorSubcoreMesh(core_axis_name="c", subcore_axis_name="s")

    @functools.partial(
        pl.kernel,
        out_type=jax.ShapeDtypeStruct((NC, n_pad, dk), jnp.float32),
        mesh=mesh,
        compiler_params=pltpu.CompilerParams(
            use_tc_tiling_on_sc=False, needs_layout_passes=False),
        scratch_types=[
            pltpu.VMEM_SHARED((n_pad, dk), jnp.float32),
            pltpu.VMEM_SHARED((n_pad, dk), jnp.float32),
            pltpu.VMEM((2, 4, EDGE_BATCH), jnp.int32),
            pltpu.VMEM((2, EDGE_BATCH, dk), jnp.float32),
            pltpu.SemaphoreType.DMA,
            pltpu.SemaphoreType.DMA,
            pltpu.SemaphoreType.DMA,
            pltpu.SemaphoreType.DMA,
        ],
    )
    def spmm(h_hbm, edata_hbm, zero_hbm, out_hbm,
             acc, hs, e_v, rows, se0, se1, sg0, sg1):
        cid = lax.axis_index("c")
        sid = lax.axis_index("s")
        wid = cid * NS + sid
        se = (se0, se1)
        sg = (sg0, sg1)
        n_rows = h_hbm.shape[0]
        last_h = n_rows - (NS - 1) * rpt

        pltpu.sync_copy(zero_hbm.at[pl.ds(sid * rpt, rpt)],
                        acc.at[pl.ds(sid * rpt, rpt)])

        @pl.when(sid < NS - 1)
        def _():
            pltpu.sync_copy(h_hbm.at[pl.ds(sid * rpt, rpt)],
                            hs.at[pl.ds(sid * rpt, rpt)])

        @pl.when(sid == NS - 1)
        def _():
            pltpu.sync_copy(h_hbm.at[pl.ds((NS - 1) * rpt, last_h)],
                            hs.at[pl.ds((NS - 1) * rpt, last_h)])

        plsc.subcore_barrier()

        def gather_start(p):
            pltpu.async_copy(hs.at[e_v.at[p].at[0]], rows.at[p], sg[p])

        def gather_wait(p):
            pltpu.make_async_copy(
                hs.at[e_v.at[p].at[0]], rows.at[p], sg[p]).wait()

        def edata_start(p, bb):
            pltpu.async_copy(edata_hbm.at[wid, bb], e_v.at[p], se[p])

        def edata_wait(p):
            pltpu.make_async_copy(
                edata_hbm.at[wid, 0], e_v.at[p], se[p]).wait()

        def step(bb, p):
            q = 1 - p
            edata_wait(q)
            gather_start(q)
            gather_wait(p)
            ep = e_v.at[p]
            rp = rows.at[p]

            @pl.loop(0, ngrp)
            def _grp(g):
                vvv = plsc.bitcast(ep[2, pl.ds(g * LANES, LANES)],
                                   jnp.float32)
                for i in range(LANES):
                    j = g * LANES + i
                    vv = vvv[i]
                    for k in range(nvec):
                        sl = pl.ds(k * LANES, LANES)
                        rp[j, sl] = rp[j, sl] * vv

            pltpu.sync_copy(rp, acc.at[ep.at[1]], add=True)
            edata_start(p, bb + 2)

        pltpu.async_copy(edata_hbm.at[wid, 0], e_v.at[0], se[0]).wait()
        gather_start(0)
        edata_start(1, 1)

        @pl.loop(0, nb, step=2)
        def _pair(b):
            step(b, 0)
            step(b + 1, 1)

        edata_wait(1)
        gather_wait(0)

        plsc.subcore_barrier()
        pltpu.sync_copy(acc.at[pl.ds(sid * rpt, rpt)],
                        out_hbm.at[cid, pl.ds(sid * rpt, rpt)])

    return spmm


def kernel(x, adj_indices, adj_values, W1, b1, gamma1, beta1,
           W2, b2, gamma2, beta2, W3, b3):
    n = x.shape[0]
    hid = W1.shape[0]
    dlast = 16
    e = adj_values.shape[0]
    group = NC * NS * EDGE_BATCH
    e_pad = ((e + group - 1) // group) * group
    pad = e_pad - e
    n_pad = ((n + NS * 8 - 1) // (NS * 8)) * (NS * 8)

    dst = jnp.concatenate([adj_indices[0], jnp.zeros((pad,), jnp.int32)])
    src = jnp.concatenate([adj_indices[1], jnp.zeros((pad,), jnp.int32)])
    val = jnp.concatenate([adj_values, jnp.zeros((pad,), jnp.float32)])
    nw = NC * NS
    epw = e_pad // nw
    nb = epw // EDGE_BATCH

    def _tile(a):
        a = a.reshape(nw, epw)
        a = jnp.concatenate(
            [a, jnp.zeros((nw, 2 * EDGE_BATCH), jnp.int32)], axis=1)
        return a.reshape(nw, nb + 2, EDGE_BATCH)

    edata = jnp.stack(
        [_tile(src), _tile(dst), _tile(lax.bitcast_convert_type(val, jnp.int32)),
         _tile(jnp.zeros((e_pad,), jnp.int32))], axis=2)

    zhid = jnp.zeros((n_pad, hid), jnp.float32)
    zlast = jnp.zeros((n_pad, dlast), jnp.float32)
    w3p = jnp.zeros((dlast, hid), jnp.float32).at[:W3.shape[0], :].set(W3)
    b3p = jnp.zeros((dlast,), jnp.float32).at[:W3.shape[0]].set(b3)

    spmm_h = _make_spmm(n_pad, hid, nb)
    spmm_l = _make_spmm(n_pad, dlast, nb)

    y1 = _mm(x, W1)
    p1 = spmm_h(y1, edata, zhid)
    y2 = _fuse(p1, W2, gamma1, beta1, n)
    p2 = spmm_h(y2, edata, zhid)
    y3 = _fuse(p2, w3p, gamma2, beta2, n)
    p3 = spmm_l(y3, edata, zlast)
    out = _final_add(p3, b3p, n)
    return out[:, :W3.shape[0]]


# speedup vs baseline: 37.2333x; 1.2563x over previous
import functools

import jax
import jax.numpy as jnp
from jax import lax
from jax.experimental import pallas as pl
from jax.experimental.pallas import tpu as pltpu
from jax.experimental.pallas import tpu_sc as plsc

NC = 2
NS = 16
LANES = 16
EDGE_BATCH = 128


def _mm(x, w):
    n, kdim = x.shape
    dout = w.shape[0]
    br = 1000

    def body(x_ref, w_ref, o_ref):
        o_ref[...] = lax.dot_general(
            x_ref[...], w_ref[...], (((1,), (1,)), ((), ())),
            preferred_element_type=jnp.float32)

    return pl.pallas_call(
        body,
        grid=(n // br,),
        in_specs=[pl.BlockSpec((br, kdim), lambda i: (i, 0)),
                  pl.BlockSpec((dout, kdim), lambda i: (0, 0))],
        out_specs=pl.BlockSpec((br, dout), lambda i: (i, 0)),
        out_shape=jax.ShapeDtypeStruct((n, dout), jnp.float32),
    )(x, w)


def _fuse(part, w, gamma, beta, n):
    dk = part.shape[2]
    dout = w.shape[0]

    def body(p_ref, w_ref, g_ref, bt_ref, o_ref):
        s = p_ref[0] + p_ref[1]
        m = jnp.mean(s, axis=0, keepdims=True)
        c = s - m
        v = jnp.mean(c * c, axis=0, keepdims=True)
        h = g_ref[...] * c * lax.rsqrt(v + 1e-5) + bt_ref[...]
        h = jnp.maximum(h, 0.0)
        o_ref[...] = lax.dot_general(
            h, w_ref[...], (((1,), (1,)), ((), ())),
            preferred_element_type=jnp.float32)

    return pl.pallas_call(
        body,
        grid=(1,),
        in_specs=[pl.BlockSpec((2, n, dk), lambda i: (0, 0, 0)),
                  pl.BlockSpec((dout, dk), lambda i: (0, 0)),
                  pl.BlockSpec((1, dk), lambda i: (0, 0)),
                  pl.BlockSpec((1, dk), lambda i: (0, 0))],
        out_specs=pl.BlockSpec((n, dout), lambda i: (0, 0)),
        out_shape=jax.ShapeDtypeStruct((n, dout), jnp.float32),
    )(part, w, gamma.reshape(1, dk), beta.reshape(1, dk))


def _final_add(part, b3p, n):
    dk = part.shape[2]

    def body(p_ref, b_ref, o_ref):
        o_ref[...] = p_ref[0] + p_ref[1] + b_ref[...]

    return pl.pallas_call(
        body,
        grid=(1,),
        in_specs=[pl.BlockSpec((2, n, dk), lambda i: (0, 0, 0)),
                  pl.BlockSpec((1, dk), lambda i: (0, 0))],
        out_specs=pl.BlockSpec((n, dk), lambda i: (0, 0)),
        out_shape=jax.ShapeDtypeStruct((n, dk), jnp.float32),
    )(part, b3p.reshape(1, dk))


@functools.lru_cache(maxsize=None)
def _make_spmm(n_pad, dk, nb):
    rpt = n_pad // NS
    nvec = dk // LANES
    ngrp = EDGE_BATCH // LANES
    NBUF = 4
    assert nb >= 2 * NBUF and nb % NBUF == 0
    mesh = plsc.VectorSubcoreMesh(core_axis_name="c", subcore_axis_name="s")

    @functools.partial(
        pl.kernel,
        out_type=jax.ShapeDtypeStruct((NC, n_pad, dk), jnp.float32),
        mesh=mesh,
        compiler_params=pltpu.CompilerParams(
            use_tc_tiling_on_sc=False, needs_layout_passes=False),
        scratch_types=[
            pltpu.VMEM_SHARED((n_pad, dk), jnp.float32),
            pltpu.VMEM_SHARED((n_pad, dk), jnp.float32),
            pltpu.VMEM((NBUF, 4, EDGE_BATCH), jnp.int32),
            pltpu.VMEM((NBUF, EDGE_BATCH, dk), jnp.float32),
            pltpu.VMEM((NBUF, EDGE_BATCH), jnp.int32),
        ] + [pltpu.SemaphoreType.DMA] * (3 * NBUF),
    )
    def spmm(h_hbm, edata_hbm, zero_hbm, out_hbm,
             acc, hs, e_v, rows, dcp, *sems):
        cid = lax.axis_index("c")
        sid = lax.axis_index("s")
        wid = cid * NS + sid
        se = sems[:NBUF]
        sg = sems[NBUF:2 * NBUF]
        ss = sems[2 * NBUF:]
        n_rows = h_hbm.shape[0]
        last_h = n_rows - (NS - 1) * rpt

        pltpu.sync_copy(zero_hbm.at[pl.ds(sid * rpt, rpt)],
                        acc.at[pl.ds(sid * rpt, rpt)])

        @pl.when(sid < NS - 1)
        def _():
            pltpu.sync_copy(h_hbm.at[pl.ds(sid * rpt, rpt)],
                            hs.at[pl.ds(sid * rpt, rpt)])

        @pl.when(sid == NS - 1)
        def _():
            pltpu.sync_copy(h_hbm.at[pl.ds((NS - 1) * rpt, last_h)],
                            hs.at[pl.ds((NS - 1) * rpt, last_h)])

        plsc.subcore_barrier()

        def gather_start(m):
            pltpu.async_copy(hs.at[e_v.at[m].at[0]], rows.at[m], sg[m])

        def gather_wait(m):
            pltpu.make_async_copy(
                hs.at[e_v.at[m].at[0]], rows.at[m], sg[m]).wait()

        def edata_start(m, bb):
            pltpu.async_copy(edata_hbm.at[wid, bb], e_v.at[m], se[m])

        def edata_wait(m):
            pltpu.make_async_copy(
                edata_hbm.at[wid, 0], e_v.at[m], se[m]).wait()

        def scat_wait(m):
            pltpu.make_async_copy(rows.at[m], acc.at[dcp.at[m]],
                                  ss[m]).wait()

        def step(bb, m, do_scat_wait=True):
            m1 = (m + 1) % NBUF
            edata_wait(m1)
            if do_scat_wait:
                scat_wait(m1)
            gather_start(m1)
            gather_wait(m)
            em = e_v.at[m]
            rm = rows.at[m]
            dm = dcp.at[m]
            for t in range(ngrp):
                sl = pl.ds(t * LANES, LANES)
                dm[sl] = em[1, sl]

            @pl.loop(0, ngrp)
            def _grp(g):
                vvv = plsc.bitcast(em[2, pl.ds(g * LANES, LANES)],
                                   jnp.float32)
                for i in range(0, LANES, 2):
                    j0 = g * LANES + i
                    j1 = g * LANES + i + 1
                    v0 = vvv[i]
                    v1 = vvv[i + 1]
                    for k in range(nvec):
                        sl = pl.ds(k * LANES, LANES)
                        a = rm[j0, sl] * v0
                        b = rm[j1, sl] * v1
                        rm[j0, sl] = a
                        rm[j1, sl] = b

            pltpu.async_copy(rm, acc.at[dm], ss[m], add=True)
            edata_start(m, bb + NBUF)

        pltpu.async_copy(edata_hbm.at[wid, 0], e_v.at[0], se[0]).wait()
        gather_start(0)
        for m in range(1, NBUF):
            edata_start(m, m)
        for bb in range(NBUF - 1):
            step(bb, bb, do_scat_wait=False)

        @pl.loop(NBUF - 1, nb - 1, step=NBUF)
        def _quad(b):
            for ph in range(NBUF):
                step(b + ph, (NBUF - 1 + ph) % NBUF)

        step(nb - 1, (nb - 1) % NBUF)

        for m in range(1, NBUF):
            edata_wait(m)
        gather_wait(0)
        for m in range(1, NBUF):
            scat_wait(m)

        plsc.subcore_barrier()
        pltpu.sync_copy(acc.at[pl.ds(sid * rpt, rpt)],
                        out_hbm.at[cid, pl.ds(sid * rpt, rpt)])

    return spmm


def kernel(x, adj_indices, adj_values, W1, b1, gamma1, beta1,
           W2, b2, gamma2, beta2, W3, b3):
    n = x.shape[0]
    hid = W1.shape[0]
    dlast = 16
    e = adj_values.shape[0]
    group = NC * NS * EDGE_BATCH
    e_pad = ((e + group - 1) // group) * group
    pad = e_pad - e
    n_pad = ((n + NS * 8 - 1) // (NS * 8)) * (NS * 8)

    dst = jnp.concatenate([adj_indices[0], jnp.zeros((pad,), jnp.int32)])
    src = jnp.concatenate([adj_indices[1], jnp.zeros((pad,), jnp.int32)])
    val = jnp.concatenate([adj_values, jnp.zeros((pad,), jnp.float32)])
    nw = NC * NS
    epw = e_pad // nw
    nb = epw // EDGE_BATCH

    def _tile(a):
        a = a.reshape(nw, epw)
        a = jnp.concatenate(
            [a, jnp.zeros((nw, 4 * EDGE_BATCH), jnp.int32)], axis=1)
        return a.reshape(nw, nb + 4, EDGE_BATCH)

    edata = jnp.stack(
        [_tile(src), _tile(dst), _tile(lax.bitcast_convert_type(val, jnp.int32)),
         _tile(jnp.zeros((e_pad,), jnp.int32))], axis=2)

    zhid = jnp.zeros((n_pad, hid), jnp.float32)
    zlast = jnp.zeros((n_pad, dlast), jnp.float32)
    w3p = jnp.zeros((dlast, hid), jnp.float32).at[:W3.shape[0], :].set(W3)
    b3p = jnp.zeros((dlast,), jnp.float32).at[:W3.shape[0]].set(b3)

    spmm_h = _make_spmm(n_pad, hid, nb)
    spmm_l = _make_spmm(n_pad, dlast, nb)

    y1 = _mm(x, W1)
    p1 = spmm_h(y1, edata, zhid)
    y2 = _fuse(p1, W2, gamma1, beta1, n)
    p2 = spmm_h(y2, edata, zhid)
    y3 = _fuse(p2, w3p, gamma2, beta2, n)
    p3 = spmm_l(y3, edata, zlast)
    out = _final_add(p3, b3p, n)
    return out[:, :W3.shape[0]]


# speedup vs baseline: 37.6550x; 1.0113x over previous
import functools

import jax
import jax.numpy as jnp
from jax import lax
from jax.experimental import pallas as pl
from jax.experimental.pallas import tpu as pltpu
from jax.experimental.pallas import tpu_sc as plsc

NC = 2
NS = 16
LANES = 16
EDGE_BATCH = 128


def _mm(x, w):
    n, kdim = x.shape
    dout = w.shape[0]
    br = 1000

    def body(x_ref, w_ref, o_ref):
        o_ref[...] = lax.dot_general(
            x_ref[...], w_ref[...], (((1,), (1,)), ((), ())),
            preferred_element_type=jnp.float32)

    return pl.pallas_call(
        body,
        grid=(n // br,),
        in_specs=[pl.BlockSpec((br, kdim), lambda i: (i, 0)),
                  pl.BlockSpec((dout, kdim), lambda i: (0, 0))],
        out_specs=pl.BlockSpec((br, dout), lambda i: (i, 0)),
        out_shape=jax.ShapeDtypeStruct((n, dout), jnp.float32),
    )(x, w)


def _fuse(part, w, gamma, beta, n):
    dk = part.shape[2]
    dout = w.shape[0]

    def body(p_ref, w_ref, g_ref, bt_ref, o_ref):
        s = p_ref[0] + p_ref[1]
        m = jnp.mean(s, axis=0, keepdims=True)
        c = s - m
        v = jnp.mean(c * c, axis=0, keepdims=True)
        h = g_ref[...] * c * lax.rsqrt(v + 1e-5) + bt_ref[...]
        h = jnp.maximum(h, 0.0)
        o_ref[...] = lax.dot_general(
            h, w_ref[...], (((1,), (1,)), ((), ())),
            preferred_element_type=jnp.float32)

    return pl.pallas_call(
        body,
        grid=(1,),
        in_specs=[pl.BlockSpec((2, n, dk), lambda i: (0, 0, 0)),
                  pl.BlockSpec((dout, dk), lambda i: (0, 0)),
                  pl.BlockSpec((1, dk), lambda i: (0, 0)),
                  pl.BlockSpec((1, dk), lambda i: (0, 0))],
        out_specs=pl.BlockSpec((n, dout), lambda i: (0, 0)),
        out_shape=jax.ShapeDtypeStruct((n, dout), jnp.float32),
    )(part, w, gamma.reshape(1, dk), beta.reshape(1, dk))


def _final_add(part, b3p, n, dout):
    dk = part.shape[2]

    def body(p_ref, b_ref, o_ref):
        o_ref[...] = (p_ref[0] + p_ref[1] + b_ref[...])[:, :dout]

    return pl.pallas_call(
        body,
        grid=(1,),
        in_specs=[pl.BlockSpec((2, n, dk), lambda i: (0, 0, 0)),
                  pl.BlockSpec((1, dk), lambda i: (0, 0))],
        out_specs=pl.BlockSpec((n, dout), lambda i: (0, 0)),
        out_shape=jax.ShapeDtypeStruct((n, dout), jnp.float32),
    )(part, b3p.reshape(1, dk))


@functools.lru_cache(maxsize=None)
def _make_spmm(n_pad, dk, nb):
    rpt = n_pad // NS
    nvec = dk // LANES
    ngrp = EDGE_BATCH // LANES
    NBUF = 4
    assert nb >= 2 * NBUF and nb % NBUF == 0
    mesh = plsc.VectorSubcoreMesh(core_axis_name="c", subcore_axis_name="s")

    @functools.partial(
        pl.kernel,
        out_type=jax.ShapeDtypeStruct((NC, n_pad, dk), jnp.float32),
        mesh=mesh,
        compiler_params=pltpu.CompilerParams(
            use_tc_tiling_on_sc=False, needs_layout_passes=False),
        scratch_types=[
            pltpu.VMEM_SHARED((n_pad, dk), jnp.float32),
            pltpu.VMEM_SHARED((n_pad, dk), jnp.float32),
            pltpu.VMEM((NBUF, 4, EDGE_BATCH), jnp.int32),
            pltpu.VMEM((NBUF, EDGE_BATCH, dk), jnp.float32),
            pltpu.VMEM((NBUF, EDGE_BATCH), jnp.int32),
        ] + [pltpu.SemaphoreType.DMA] * (3 * NBUF),
    )
    def spmm(h_hbm, edata_hbm, out_hbm,
             acc, hs, e_v, rows, dcp, *sems):
        cid = lax.axis_index("c")
        sid = lax.axis_index("s")
        wid = cid * NS + sid
        se = sems[:NBUF]
        sg = sems[NBUF:2 * NBUF]
        ss = sems[2 * NBUF:]
        n_rows = h_hbm.shape[0]
        last_h = n_rows - (NS - 1) * rpt

        zb = rows.at[0]

        @pl.loop(0, EDGE_BATCH)
        def _zero(r):
            for k in range(nvec):
                zb[r, pl.ds(k * LANES, LANES)] = jnp.zeros(
                    (LANES,), jnp.float32)

        for c in range(rpt // EDGE_BATCH):
            pltpu.sync_copy(
                zb, acc.at[pl.ds(sid * rpt + c * EDGE_BATCH, EDGE_BATCH)])

        @pl.when(sid < NS - 1)
        def _():
            pltpu.sync_copy(h_hbm.at[pl.ds(sid * rpt, rpt)],
                            hs.at[pl.ds(sid * rpt, rpt)])

        @pl.when(sid == NS - 1)
        def _():
            pltpu.sync_copy(h_hbm.at[pl.ds((NS - 1) * rpt, last_h)],
                            hs.at[pl.ds((NS - 1) * rpt, last_h)])

        plsc.subcore_barrier()

        def gather_start(m):
            pltpu.async_copy(hs.at[e_v.at[m].at[0]], rows.at[m], sg[m])

        def gather_wait(m):
            pltpu.make_async_copy(
                hs.at[e_v.at[m].at[0]], rows.at[m], sg[m]).wait()

        def edata_start(m, bb):
            pltpu.async_copy(edata_hbm.at[wid, bb], e_v.at[m], se[m])

        def edata_wait(m):
            pltpu.make_async_copy(
                edata_hbm.at[wid, 0], e_v.at[m], se[m]).wait()

        def scat_wait(m):
            pltpu.make_async_copy(rows.at[m], acc.at[dcp.at[m]],
                                  ss[m]).wait()

        def step(bb, m, do_scat_wait=True):
            m1 = (m + 1) % NBUF
            edata_wait(m1)
            if do_scat_wait:
                scat_wait(m1)
            gather_start(m1)
            gather_wait(m)
            em = e_v.at[m]
            rm = rows.at[m]
            dm = dcp.at[m]
            for t in range(ngrp):
                sl = pl.ds(t * LANES, LANES)
                dm[sl] = em[1, sl]

            @pl.loop(0, ngrp)
            def _grp(g):
                vvv = plsc.bitcast(em[2, pl.ds(g * LANES, LANES)],
                                   jnp.float32)
                for i in range(0, LANES, 2):
                    j0 = g * LANES + i
                    j1 = g * LANES + i + 1
                    v0 = vvv[i]
                    v1 = vvv[i + 1]
                    for k in range(nvec):
                        sl = pl.ds(k * LANES, LANES)
                        a = rm[j0, sl] * v0
                        b = rm[j1, sl] * v1
                        rm[j0, sl] = a
                        rm[j1, sl] = b

            pltpu.async_copy(rm, acc.at[dm], ss[m], add=True)
            edata_start(m, bb + NBUF)

        pltpu.async_copy(edata_hbm.at[wid, 0], e_v.at[0], se[0]).wait()
        gather_start(0)
        for m in range(1, NBUF):
            edata_start(m, m)
        for bb in range(NBUF - 1):
            step(bb, bb, do_scat_wait=False)

        @pl.loop(NBUF - 1, nb - 1, step=NBUF)
        def _quad(b):
            for ph in range(NBUF):
                step(b + ph, (NBUF - 1 + ph) % NBUF)

        step(nb - 1, (nb - 1) % NBUF)

        for m in range(1, NBUF):
            edata_wait(m)
        gather_wait(0)
        for m in range(1, NBUF):
            scat_wait(m)

        plsc.subcore_barrier()
        pltpu.sync_copy(acc.at[pl.ds(sid * rpt, rpt)],
                        out_hbm.at[cid, pl.ds(sid * rpt, rpt)])

    return spmm


def kernel(x, adj_indices, adj_values, W1, b1, gamma1, beta1,
           W2, b2, gamma2, beta2, W3, b3):
    n = x.shape[0]
    hid = W1.shape[0]
    dlast = 16
    e = adj_values.shape[0]
    group = NC * NS * EDGE_BATCH
    e_pad = ((e + group - 1) // group) * group
    pad = e_pad - e
    n_pad = ((n + NS * 8 - 1) // (NS * 8)) * (NS * 8)

    dst = jnp.concatenate([adj_indices[0], jnp.zeros((pad,), jnp.int32)])
    src = jnp.concatenate([adj_indices[1], jnp.zeros((pad,), jnp.int32)])
    val = jnp.concatenate([adj_values, jnp.zeros((pad,), jnp.float32)])
    nw = NC * NS
    epw = e_pad // nw
    nb = epw // EDGE_BATCH

    def _tile(a):
        a = a.reshape(nw, epw)
        a = jnp.concatenate(
            [a, jnp.zeros((nw, 4 * EDGE_BATCH), jnp.int32)], axis=1)
        return a.reshape(nw, nb + 4, EDGE_BATCH)

    edata = jnp.stack(
        [_tile(src), _tile(dst), _tile(lax.bitcast_convert_type(val, jnp.int32)),
         _tile(jnp.zeros((e_pad,), jnp.int32))], axis=2)

    w3p = jnp.zeros((dlast, hid), jnp.float32).at[:W3.shape[0], :].set(W3)
    b3p = jnp.zeros((dlast,), jnp.float32).at[:W3.shape[0]].set(b3)

    spmm_h = _make_spmm(n_pad, hid, nb)
    spmm_l = _make_spmm(n_pad, dlast, nb)

    y1 = _mm(x, W1)
    p1 = spmm_h(y1, edata)
    y2 = _fuse(p1, W2, gamma1, beta1, n)
    p2 = spmm_h(y2, edata)
    y3 = _fuse(p2, w3p, gamma2, beta2, n)
    p3 = spmm_l(y3, edata)
    return _final_add(p3, b3p, n, W3.shape[0])
